# baseline (speedup 1.0000x reference)
import jax
import jax.numpy as jnp
from jax.experimental import pallas as pl
from jax.experimental.pallas import tpu as pltpu

B, C, H, W = 8, 320, 48, 48
HW = H * W


def _scale_body(n_ref, g_ref, x_ref, o_ref):
    g = jnp.abs(g_ref[0])
    o_ref[...] = g[:, :, None] * x_ref[...]


def kernel(x, n, gain_matrix):
    xf = x.reshape(B, C, HW)
    g3 = gain_matrix.reshape(B, 1, C)
    out = pl.pallas_call(
        _scale_body,
        grid_spec=pltpu.PrefetchScalarGridSpec(
            num_scalar_prefetch=1,
            grid=(B,),
            in_specs=[
                pl.BlockSpec((1, 1, C), lambda b, n_ref: (n_ref[b], 0, 0)),
                pl.BlockSpec((1, C, HW), lambda b, n_ref: (b, 0, 0)),
            ],
            out_specs=pl.BlockSpec((1, C, HW), lambda b, n_ref: (b, 0, 0)),
        ),
        out_shape=jax.ShapeDtypeStruct((B, C, HW), jnp.float32),
    )(n.astype(jnp.int32), g3, xf)
    return out.reshape(B, C, H, W)
